# baseline (speedup 1.0000x reference)
import functools

import jax
import jax.numpy as jnp
from jax import lax
from jax.experimental import pallas as pl
from jax.experimental.pallas import tpu as pltpu
from jax.experimental.pallas import tpu_sc as plsc

NUM_EMB = 1000000
DIM = 32
BATCH = 16384
GRP = 128 // DIM
NGRP = NUM_EMB // GRP

NC = 2
NS = 16
NW = NC * NS
BPW = BATCH // NW
IDX_MINOR = 128
NJ = BPW // IDX_MINOR


def _sc_gather_body(xg_hbm, table_hbm, out_hbm, idx_v, grp_v, sem):
    wid = lax.axis_index("s") * NC + lax.axis_index("c")

    pltpu.sync_copy(xg_hbm.at[wid], idx_v)

    copies = [
        pltpu.async_copy(
            table_hbm.at[idx_v.at[j]],
            grp_v.at[pl.ds(j * IDX_MINOR, IDX_MINOR)],
            sem,
        )
        for j in range(NJ)
    ]
    for c in copies:
        c.wait()

    pltpu.sync_copy(grp_v, out_hbm.at[wid])


TC_BLK = 2048
TC_GRID = BATCH // TC_BLK


def _tc_dense_body(grp_ref, wbt_ref, xm_ref, b_ref, out_ref):
    grp = grp_ref[...]
    wbt = wbt_ref[...]
    acc4 = jnp.dot(grp, wbt, preferred_element_type=jnp.float32)
    sel = lax.broadcasted_iota(jnp.int32, acc4.shape, 1) == xm_ref[...]
    acc = jnp.sum(jnp.where(sel, acc4, 0.0), axis=1, keepdims=True)
    acc = acc + b_ref[0, 0]
    out_ref[...] = 1.0 / (1.0 + jnp.exp(-acc))


@jax.jit
def _run(xg3, xm2, table2, wbt, b2):
    mesh = plsc.VectorSubcoreMesh(core_axis_name="c", subcore_axis_name="s")
    gather = functools.partial(
        pl.kernel,
        mesh=mesh,
        out_type=jax.ShapeDtypeStruct((NW, BPW, 128), jnp.float32),
        scratch_types=[
            pltpu.VMEM((NJ, IDX_MINOR), jnp.int32),
            pltpu.VMEM((BPW, 128), jnp.float32),
            pltpu.SemaphoreType.DMA,
        ],
    )(_sc_gather_body)
    groups = gather(xg3, table2).reshape(BATCH, 128)

    dense = pl.pallas_call(
        _tc_dense_body,
        grid=(TC_GRID,),
        in_specs=[
            pl.BlockSpec((TC_BLK, 128), lambda i: (i, 0)),
            pl.BlockSpec((128, GRP), lambda i: (0, 0)),
            pl.BlockSpec((TC_BLK, 1), lambda i: (i, 0)),
            pl.BlockSpec(memory_space=pltpu.SMEM),
        ],
        out_specs=pl.BlockSpec((TC_BLK, 1), lambda i: (i, 0)),
        out_shape=jax.ShapeDtypeStruct((BATCH, 1), jnp.float32),
    )
    return dense(groups, wbt, xm2, b2)


def kernel(x, table, W, b):
    xi = x.astype(jnp.int32)
    xg3 = (xi // GRP).reshape(NW, NJ, IDX_MINOR)
    xm2 = (xi % GRP).reshape(BATCH, 1)
    table2 = table.reshape(NGRP, 128)
    wcol = W.reshape(DIM)
    wbt = (jnp.eye(GRP, dtype=jnp.float32)[:, None, :]
           * wcol[None, :, None]).reshape(GRP * DIM, GRP)
    b2 = b.reshape(1, 1)
    return _run(xg3, xm2, table2, wbt, b2)


# speedup vs baseline: 1.6527x; 1.6527x over previous
import functools

import jax
import jax.numpy as jnp
from jax import lax
from jax.experimental import pallas as pl
from jax.experimental.pallas import tpu as pltpu
from jax.experimental.pallas import tpu_sc as plsc

NUM_EMB = 1000000
DIM = 32
BATCH = 16384

NC = 2
NS = 16
NW = NC * NS
BPW = BATCH // NW


def _sc_gather_body(x_hbm, table_hbm, out_hbm, idx_v, rows_v, sem):
    wid = lax.axis_index("s") * NC + lax.axis_index("c")

    pltpu.sync_copy(x_hbm.at[wid], idx_v)

    def fire(c, carry):
        vec = idx_v[pl.ds(c * 16, 16)]
        base = c * 16
        for j in range(16):
            pltpu.async_copy(table_hbm.at[vec[j]], rows_v.at[base + j], sem)
        return carry

    lax.fori_loop(0, BPW // 16, fire, 0)

    pltpu.make_async_copy(table_hbm.at[pl.ds(0, BPW)], rows_v, sem).wait()

    pltpu.sync_copy(rows_v, out_hbm.at[wid])


TC_BLK = 2048
TC_GRID = BATCH // TC_BLK


def _tc_dense_body(rows_ref, wt_ref, b_ref, out_ref):
    rows = rows_ref[...]
    wt = wt_ref[...]
    acc = jnp.sum(rows * wt, axis=1, keepdims=True) + b_ref[0, 0]
    out_ref[...] = 1.0 / (1.0 + jnp.exp(-acc))


@jax.jit
def _run(x2, table, wt, b2):
    mesh = plsc.VectorSubcoreMesh(core_axis_name="c", subcore_axis_name="s")
    gather = functools.partial(
        pl.kernel,
        mesh=mesh,
        out_type=jax.ShapeDtypeStruct((NW, BPW, DIM), jnp.float32),
        scratch_types=[
            pltpu.VMEM((BPW,), jnp.int32),
            pltpu.VMEM((BPW, DIM), jnp.float32),
            pltpu.SemaphoreType.DMA,
        ],
    )(_sc_gather_body)
    rows = gather(x2, table).reshape(BATCH, DIM)

    dense = pl.pallas_call(
        _tc_dense_body,
        grid=(TC_GRID,),
        in_specs=[
            pl.BlockSpec((TC_BLK, DIM), lambda i: (i, 0)),
            pl.BlockSpec((1, DIM), lambda i: (0, 0)),
            pl.BlockSpec(memory_space=pltpu.SMEM),
        ],
        out_specs=pl.BlockSpec((TC_BLK, 1), lambda i: (i, 0)),
        out_shape=jax.ShapeDtypeStruct((BATCH, 1), jnp.float32),
    )
    return dense(rows, wt, b2)


def kernel(x, table, W, b):
    x2 = x.astype(jnp.int32).reshape(NW, BPW)
    wt = W.reshape(1, DIM)
    b2 = b.reshape(1, 1)
    return _run(x2, table, wt, b2)


# speedup vs baseline: 6.1191x; 3.7025x over previous
import functools

import jax
import jax.numpy as jnp
from jax import lax
from jax.experimental import pallas as pl
from jax.experimental.pallas import tpu as pltpu
from jax.experimental.pallas import tpu_sc as plsc

NUM_EMB = 1000000
DIM = 32
BATCH = 16384

NC = 2
NS = 16
NW = NC * NS
BPW = BATCH // NW
IDX_MINOR = 128
NJ = BPW // IDX_MINOR

TV_PAD = 1 << 20
A_BLK = 65536
A_GRID = TV_PAD // A_BLK
TVR = TV_PAD // 128


def _tv_body(wt_ref, tbl_ref, tv_ref):
    tv_ref[...] = jnp.dot(
        wt_ref[...], tbl_ref[...], preferred_element_type=jnp.float32
    )


def _sc_gather_body(xg_hbm, tv_hbm, out_hbm, idx_v, rows_v, sem):
    wid = lax.axis_index("s") * NC + lax.axis_index("c")

    pltpu.sync_copy(xg_hbm.at[wid], idx_v)

    copies = [
        pltpu.async_copy(
            tv_hbm.at[idx_v.at[j]],
            rows_v.at[pl.ds(j * IDX_MINOR, IDX_MINOR)],
            sem,
        )
        for j in range(NJ)
    ]
    for c in copies:
        c.wait()

    pltpu.sync_copy(rows_v, out_hbm.at[wid])


TC_BLK = 2048
TC_GRID = BATCH // TC_BLK


def _sel_body(grp_ref, xm_ref, b_ref, out_ref):
    grp = grp_ref[...]
    oh = lax.broadcasted_iota(jnp.int32, grp.shape, 1) == xm_ref[...]
    acc = jnp.sum(jnp.where(oh, grp, 0.0), axis=1, keepdims=True)
    acc = acc + b_ref[0, 0]
    out_ref[...] = 1.0 / (1.0 + jnp.exp(-acc))


@jax.jit
def _run(xg3, xm2, tableT, wt, b2):
    tv = pl.pallas_call(
        _tv_body,
        grid=(A_GRID,),
        in_specs=[
            pl.BlockSpec((1, DIM), lambda j: (0, 0)),
            pl.BlockSpec((DIM, A_BLK), lambda j: (0, j)),
        ],
        out_specs=pl.BlockSpec((1, A_BLK), lambda j: (0, j)),
        out_shape=jax.ShapeDtypeStruct((1, TV_PAD), jnp.float32),
    )(wt, tableT)
    tv2 = tv.reshape(TVR, 128)

    mesh = plsc.VectorSubcoreMesh(core_axis_name="c", subcore_axis_name="s")
    gather = functools.partial(
        pl.kernel,
        mesh=mesh,
        out_type=jax.ShapeDtypeStruct((NW, BPW, 128), jnp.float32),
        scratch_types=[
            pltpu.VMEM((NJ, IDX_MINOR), jnp.int32),
            pltpu.VMEM((BPW, 128), jnp.float32),
            pltpu.SemaphoreType.DMA,
        ],
    )(_sc_gather_body)
    groups = gather(xg3, tv2).reshape(BATCH, 128)

    return pl.pallas_call(
        _sel_body,
        grid=(TC_GRID,),
        in_specs=[
            pl.BlockSpec((TC_BLK, 128), lambda i: (i, 0)),
            pl.BlockSpec((TC_BLK, 1), lambda i: (i, 0)),
            pl.BlockSpec(memory_space=pltpu.SMEM),
        ],
        out_specs=pl.BlockSpec((TC_BLK, 1), lambda i: (i, 0)),
        out_shape=jax.ShapeDtypeStruct((BATCH, 1), jnp.float32),
    )(groups, xm2, b2)


def kernel(x, table, W, b):
    xi = x.astype(jnp.int32)
    xg3 = (xi // 128).reshape(NW, NJ, IDX_MINOR)
    xm2 = (xi % 128).reshape(BATCH, 1)
    wt = W.reshape(1, DIM)
    b2 = b.reshape(1, 1)
    return _run(xg3, xm2, table.T, wt, b2)


# speedup vs baseline: 6.1282x; 1.0015x over previous
import functools

import jax
import jax.numpy as jnp
from jax import lax
from jax.experimental import pallas as pl
from jax.experimental.pallas import tpu as pltpu
from jax.experimental.pallas import tpu_sc as plsc

NUM_EMB = 1000000
DIM = 32
BATCH = 16384

NC = 2
NS = 16
NW = NC * NS
BPW = BATCH // NW
IDX_MINOR = 128
NJ = BPW // IDX_MINOR

TV_PAD = 1 << 20
A_BLK = 65536
A_GRID = TV_PAD // A_BLK
TVR = TV_PAD // 128


def _tv_body(wt_ref, tbl_ref, tv_ref):
    tv_ref[...] = jnp.dot(
        wt_ref[...], tbl_ref[...], preferred_element_type=jnp.float32
    ).reshape(A_BLK)


def _sc_gather_body(xg_hbm, tv_hbm, out_hbm, idx_v, rows_v, sem):
    wid = lax.axis_index("s") * NC + lax.axis_index("c")

    pltpu.sync_copy(xg_hbm.at[wid], idx_v)

    copies = [
        pltpu.async_copy(
            tv_hbm.at[idx_v.at[j]],
            rows_v.at[pl.ds(j * IDX_MINOR, IDX_MINOR)],
            sem,
        )
        for j in range(NJ)
    ]
    for c in copies:
        c.wait()

    pltpu.sync_copy(rows_v, out_hbm.at[pl.ds(wid * BPW, BPW)])


TC_BLK = 2048
TC_GRID = BATCH // TC_BLK


def _sel_body(grp_ref, xm_ref, b_ref, out_ref):
    grp = grp_ref[...]
    oh = lax.broadcasted_iota(jnp.int32, grp.shape, 1) == xm_ref[...]
    acc = jnp.sum(jnp.where(oh, grp, 0.0), axis=1, keepdims=True)
    acc = acc + b_ref[0, 0]
    out_ref[...] = 1.0 / (1.0 + jnp.exp(-acc))


@jax.jit
def _run(xg3, xm2, tableT, wt, b2):
    tv = pl.pallas_call(
        _tv_body,
        grid=(A_GRID,),
        in_specs=[
            pl.BlockSpec((1, DIM), lambda j: (0, 0)),
            pl.BlockSpec((DIM, A_BLK), lambda j: (0, j)),
        ],
        out_specs=pl.BlockSpec((A_BLK,), lambda j: (j,)),
        out_shape=jax.ShapeDtypeStruct((TV_PAD,), jnp.float32),
    )(wt, tableT)
    tv2 = tv.reshape(TVR, 128)

    mesh = plsc.VectorSubcoreMesh(core_axis_name="c", subcore_axis_name="s")
    gather = functools.partial(
        pl.kernel,
        mesh=mesh,
        out_type=jax.ShapeDtypeStruct((BATCH, 128), jnp.float32),
        scratch_types=[
            pltpu.VMEM((NJ, IDX_MINOR), jnp.int32),
            pltpu.VMEM((BPW, 128), jnp.float32),
            pltpu.SemaphoreType.DMA,
        ],
    )(_sc_gather_body)
    groups = gather(xg3, tv2)

    return pl.pallas_call(
        _sel_body,
        grid=(TC_GRID,),
        in_specs=[
            pl.BlockSpec((TC_BLK, 128), lambda i: (i, 0)),
            pl.BlockSpec((TC_BLK, 1), lambda i: (i, 0)),
            pl.BlockSpec(memory_space=pltpu.SMEM),
        ],
        out_specs=pl.BlockSpec((TC_BLK, 1), lambda i: (i, 0)),
        out_shape=jax.ShapeDtypeStruct((BATCH, 1), jnp.float32),
    )(groups, xm2, b2)


def kernel(x, table, W, b):
    xi = x.astype(jnp.int32)
    xg3 = (xi // 128).reshape(NW, NJ, IDX_MINOR)
    xm2 = (xi % 128).reshape(BATCH, 1)
    wt = W.reshape(1, DIM)
    b2 = b.reshape(1, 1)
    return _run(xg3, xm2, table.T, wt, b2)


# speedup vs baseline: 6.5395x; 1.0671x over previous
import functools

import jax
import jax.numpy as jnp
from jax import lax
from jax.experimental import pallas as pl
from jax.experimental.pallas import tpu as pltpu
from jax.experimental.pallas import tpu_sc as plsc

NUM_EMB = 1000000
DIM = 32
BATCH = 16384

NC = 2
NS = 16
NW = NC * NS
BPW = BATCH // NW
IDX_MINOR = 128
NJ = BPW // IDX_MINOR

TV_PAD = 1 << 20
A_BLK = 65536
A_GRID = TV_PAD // A_BLK
TVR = TV_PAD // 128


def _tv_body(wt_ref, tbl_ref, tv_ref):
    tv_ref[...] = jnp.dot(
        wt_ref[...], tbl_ref[...], preferred_element_type=jnp.float32
    ).reshape(A_BLK)


def _sc_gather_body(xg_hbm, tv_hbm, out_hbm, idx_v, rows_v, sem):
    wid = lax.axis_index("s") * NC + lax.axis_index("c")

    pltpu.sync_copy(xg_hbm.at[wid], idx_v)

    copies = [
        pltpu.async_copy(
            tv_hbm.at[idx_v.at[j]],
            rows_v.at[pl.ds(j * IDX_MINOR, IDX_MINOR)],
            sem,
        )
        for j in range(NJ)
    ]
    for c in copies:
        c.wait()

    pltpu.sync_copy(rows_v, out_hbm.at[pl.ds(wid * BPW, BPW)])


TC_BLK = 2048
TC_GRID = BATCH // TC_BLK


def _sel_body(grp_ref, xm_ref, b_ref, out_ref):
    grp = grp_ref[...]
    sel = jnp.take_along_axis(grp, xm_ref[...], axis=1)
    acc = sel.reshape(TC_BLK) + b_ref[0, 0]
    out_ref[...] = 1.0 / (1.0 + jnp.exp(-acc))


@jax.jit
def _run(xg3, xm2, tableT, wt, b2):
    tv = pl.pallas_call(
        _tv_body,
        grid=(A_GRID,),
        in_specs=[
            pl.BlockSpec((1, DIM), lambda j: (0, 0)),
            pl.BlockSpec((DIM, A_BLK), lambda j: (0, j)),
        ],
        out_specs=pl.BlockSpec((A_BLK,), lambda j: (j,)),
        out_shape=jax.ShapeDtypeStruct((TV_PAD,), jnp.float32),
    )(wt, tableT)
    tv2 = tv.reshape(TVR, 128)

    mesh = plsc.VectorSubcoreMesh(core_axis_name="c", subcore_axis_name="s")
    gather = functools.partial(
        pl.kernel,
        mesh=mesh,
        out_type=jax.ShapeDtypeStruct((BATCH, 128), jnp.float32),
        scratch_types=[
            pltpu.VMEM((NJ, IDX_MINOR), jnp.int32),
            pltpu.VMEM((BPW, 128), jnp.float32),
            pltpu.SemaphoreType.DMA,
        ],
    )(_sc_gather_body)
    groups = gather(xg3, tv2)

    return pl.pallas_call(
        _sel_body,
        grid=(TC_GRID,),
        in_specs=[
            pl.BlockSpec((TC_BLK, 128), lambda i: (i, 0)),
            pl.BlockSpec((TC_BLK, 1), lambda i: (i, 0)),
            pl.BlockSpec(memory_space=pltpu.SMEM),
        ],
        out_specs=pl.BlockSpec((TC_BLK,), lambda i: (i,)),
        out_shape=jax.ShapeDtypeStruct((BATCH,), jnp.float32),
    )(groups, xm2, b2)


def kernel(x, table, W, b):
    xi = x.astype(jnp.int32)
    xg3 = (xi // 128).reshape(NW, NJ, IDX_MINOR)
    xm2 = (xi % 128).reshape(BATCH, 1)
    wt = W.reshape(1, DIM)
    b2 = b.reshape(1, 1)
    return _run(xg3, xm2, table.T, wt, b2).reshape(BATCH, 1)


# speedup vs baseline: 7.9304x; 1.2127x over previous
import functools

import jax
import jax.numpy as jnp
from jax import lax
from jax.experimental import pallas as pl
from jax.experimental.pallas import tpu as pltpu
from jax.experimental.pallas import tpu_sc as plsc

NUM_EMB = 1000000
DIM = 32
BATCH = 16384

NC = 2
NS = 16
NW = NC * NS
BPW = BATCH // NW
IDX_MINOR = 128
NJ = BPW // IDX_MINOR
NCH = BPW // 16

TV_PAD = 1 << 20
A_BLK = 65536
A_GRID = TV_PAD // A_BLK
TVR = TV_PAD // 128


def _tv_body(wt_ref, tbl_ref, tv_ref):
    tv_ref[...] = jnp.dot(
        wt_ref[...], tbl_ref[...], preferred_element_type=jnp.float32
    ).reshape(A_BLK)


def _splat(vec16, lane):
    dnums = lax.GatherDimensionNumbers(
        offset_dims=(), collapsed_slice_dims=(0,), start_index_map=(0,)
    )
    idx = jnp.full((16, 1), lane, jnp.int32)
    return lax.gather(
        vec16, idx, dnums, slice_sizes=(1,),
        mode=lax.GatherScatterMode.PROMISE_IN_BOUNDS,
    )


def _sc_body(x_hbm, tv_hbm, b_hbm, out_hbm, xr_v, idx_v, rows_v, out_v, b_v, sem):
    wid = lax.axis_index("s") * NC + lax.axis_index("c")
    base = wid * BPW

    pltpu.sync_copy(x_hbm.at[wid], xr_v)
    pltpu.sync_copy(b_hbm, b_v)

    def shift(c, carry):
        off = pl.multiple_of(c * 16, 16)
        idx_v[pl.ds(off, 16)] = lax.shift_right_logical(
            xr_v[pl.ds(off, 16)], 7
        )
        return carry

    lax.fori_loop(0, NCH, shift, 0)

    copies = [
        pltpu.async_copy(
            tv_hbm.at[idx_v.at[pl.ds(j * IDX_MINOR, IDX_MINOR)]],
            rows_v.at[pl.ds(j * IDX_MINOR, IDX_MINOR)],
            sem,
        )
        for j in range(NJ)
    ]
    for c in copies:
        c.wait()

    bias = _splat(b_v[pl.ds(0, 16)], 0)
    lane_iota = lax.iota(jnp.int32, 16)

    def select(c, carry):
        coff = pl.multiple_of(c * 16, 16)
        xm = xr_v[pl.ds(coff, 16)] & 127
        res = bias
        for j in range(16):
            xj = xm[j]
            woff = pl.multiple_of(
                jnp.minimum(xj & ~jnp.int32(7), 112), 8
            )
            vals = rows_v[coff + j, pl.ds(woff, 16)]
            g = _splat(vals, xj - woff)
            res = jnp.where(lane_iota == j, g + bias, res)
        out_v[pl.ds(coff, 16)] = 1.0 / (1.0 + jnp.exp(-res))
        return carry

    lax.fori_loop(0, NCH, select, 0)

    pltpu.sync_copy(out_v, out_hbm.at[pl.ds(base, BPW)])


@jax.jit
def _run(x2, tableT, wt, b1):
    tv = pl.pallas_call(
        _tv_body,
        grid=(A_GRID,),
        in_specs=[
            pl.BlockSpec((1, DIM), lambda j: (0, 0)),
            pl.BlockSpec((DIM, A_BLK), lambda j: (0, j)),
        ],
        out_specs=pl.BlockSpec((A_BLK,), lambda j: (j,)),
        out_shape=jax.ShapeDtypeStruct((TV_PAD,), jnp.float32),
    )(wt, tableT)
    tv2 = tv.reshape(TVR, 128)

    mesh = plsc.VectorSubcoreMesh(core_axis_name="c", subcore_axis_name="s")
    fused = functools.partial(
        pl.kernel,
        mesh=mesh,
        out_type=jax.ShapeDtypeStruct((BATCH,), jnp.float32),
        scratch_types=[
            pltpu.VMEM((BPW,), jnp.int32),
            pltpu.VMEM((BPW,), jnp.int32),
            pltpu.VMEM((BPW, 128), jnp.float32),
            pltpu.VMEM((BPW,), jnp.float32),
            pltpu.VMEM((16,), jnp.float32),
            pltpu.SemaphoreType.DMA,
        ],
    )(_sc_body)
    return fused(x2, tv2, b1)


def kernel(x, table, W, b):
    x2 = x.astype(jnp.int32).reshape(NW, BPW)
    wt = W.reshape(1, DIM)
    b1 = jnp.concatenate([b.reshape(1), jnp.zeros((15,), jnp.float32)])
    return _run(x2, table.T, wt, b1).reshape(BATCH, 1)


# speedup vs baseline: 8.3782x; 1.0565x over previous
import functools

import jax
import jax.numpy as jnp
from jax import lax
from jax.experimental import pallas as pl
from jax.experimental.pallas import tpu as pltpu
from jax.experimental.pallas import tpu_sc as plsc

NUM_EMB = 1000000
DIM = 32
BATCH = 16384

NC = 2
NS = 16
NW = NC * NS
BPW = BATCH // NW
IDX_MINOR = 128
NJ = BPW // IDX_MINOR
NCH = BPW // 16

TV_PAD = 1 << 20
A_BLK = 65536
A_GRID = TV_PAD // A_BLK
TVR = TV_PAD // 128


def _tv_body(wt_ref, tbl_ref, tv_ref):
    tv_ref[...] = jnp.dot(
        wt_ref[...], tbl_ref[...], preferred_element_type=jnp.float32
    ).reshape(A_BLK)


def _splat(vec16, lane):
    dnums = lax.GatherDimensionNumbers(
        offset_dims=(), collapsed_slice_dims=(0,), start_index_map=(0,)
    )
    idx = jnp.full((16, 1), lane, jnp.int32)
    return lax.gather(
        vec16, idx, dnums, slice_sizes=(1,),
        mode=lax.GatherScatterMode.PROMISE_IN_BOUNDS,
    )


def _sc_body(x_hbm, tv_hbm, b_hbm, out_hbm, xr_v, idx_v, rows_v, out_v, b_v, sem):
    wid = lax.axis_index("s") * NC + lax.axis_index("c")
    base = wid * BPW

    pltpu.sync_copy(x_hbm.at[pl.ds(base, BPW)], xr_v)
    pltpu.sync_copy(b_hbm, b_v.at[pl.ds(0, 1)])

    def shift(c, carry):
        off = pl.multiple_of(c * 16, 16)
        idx_v[pl.ds(off, 16)] = lax.shift_right_logical(
            xr_v[pl.ds(off, 16)], 7
        )
        return carry

    lax.fori_loop(0, NCH, shift, 0)

    copies = [
        pltpu.async_copy(
            tv_hbm.at[idx_v.at[pl.ds(j * IDX_MINOR, IDX_MINOR)]],
            rows_v.at[pl.ds(j * IDX_MINOR, IDX_MINOR)],
            sem,
        )
        for j in range(NJ)
    ]
    for c in copies:
        c.wait()

    bias = _splat(b_v[pl.ds(0, 16)], 0)
    lane_iota = lax.iota(jnp.int32, 16)

    def select(c, carry):
        coff = pl.multiple_of(c * 16, 16)
        xm = xr_v[pl.ds(coff, 16)] & 127
        res = bias
        for j in range(16):
            xj = xm[j]
            woff = pl.multiple_of(
                jnp.minimum(xj & ~jnp.int32(7), 112), 8
            )
            vals = rows_v[coff + j, pl.ds(woff, 16)]
            g = _splat(vals, xj - woff)
            res = jnp.where(lane_iota == j, g + bias, res)
        out_v[pl.ds(coff, 16)] = 1.0 / (1.0 + jnp.exp(-res))
        return carry

    lax.fori_loop(0, NCH, select, 0)

    pltpu.sync_copy(out_v, out_hbm.at[pl.ds(base, BPW)])


@jax.jit
def _run(x1, tableT, wt, b1):
    tv = pl.pallas_call(
        _tv_body,
        grid=(A_GRID,),
        in_specs=[
            pl.BlockSpec((1, DIM), lambda j: (0, 0)),
            pl.BlockSpec((DIM, A_BLK), lambda j: (0, j)),
        ],
        out_specs=pl.BlockSpec((A_BLK,), lambda j: (j,)),
        out_shape=jax.ShapeDtypeStruct((TV_PAD,), jnp.float32),
    )(wt, tableT)
    tv2 = tv.reshape(TVR, 128)

    mesh = plsc.VectorSubcoreMesh(core_axis_name="c", subcore_axis_name="s")
    fused = functools.partial(
        pl.kernel,
        mesh=mesh,
        out_type=jax.ShapeDtypeStruct((BATCH,), jnp.float32),
        scratch_types=[
            pltpu.VMEM((BPW,), jnp.int32),
            pltpu.VMEM((BPW,), jnp.int32),
            pltpu.VMEM((BPW, 128), jnp.float32),
            pltpu.VMEM((BPW,), jnp.float32),
            pltpu.VMEM((16,), jnp.float32),
            pltpu.SemaphoreType.DMA,
        ],
    )(_sc_body)
    return fused(x1, tv2, b1)


def kernel(x, table, W, b):
    x1 = x.astype(jnp.int32)
    wt = W.reshape(1, DIM)
    return _run(x1, table.T, wt, b.reshape(1)).reshape(BATCH, 1)
